# baseline (speedup 1.0000x reference)
import jax
import jax.numpy as jnp
from jax.experimental import pallas as pl
from jax.experimental.pallas import tpu as pltpu

TEMPERATURE = 0.5
QUEUE_SIZE = 125
CLASS_NUM = 3
PROJ_DIM = 128
BATCH = 1024
TOTAL_Q = QUEUE_SIZE * CLASS_NUM
LOG2E = 1.4426950408889634
LN2 = 0.6931471805599453
HALF = BATCH // 2


def _half_loss(act, ema, pl_probs, queue):
    def dot_bt(x, y):
        return jax.lax.dot_general(x, y, (((1,), (1,)), ((), ())),
                                   preferred_element_type=jnp.float32)

    ones_k = jnp.ones((1, PROJ_DIM), dtype=jnp.float32)
    eps2 = 1e-24

    n2 = jnp.maximum(dot_bt(ones_k, act * act), eps2)
    ne2 = jnp.maximum(dot_bt(ones_k, ema * ema), eps2)
    s_ae = dot_bt(ones_k, act * ema)

    scale = LOG2E / TEMPERATURE
    rn = jax.lax.rsqrt(n2)
    tau_p = (scale * s_ae) * rn * jax.lax.rsqrt(ne2)
    l_pos = jax.lax.exp2(tau_p)

    tau = dot_bt(queue, act) * (scale * rn)
    sims = jax.lax.exp2(tau)

    total = jnp.sum(sims, axis=0, keepdims=True)
    d = l_pos + total
    a = 1e-6 * d

    plt = dot_bt(jnp.eye(CLASS_NUM, dtype=jnp.float32), pl_probs)
    p0 = plt[0:1, :]
    p1 = plt[1:2, :]
    p2 = plt[2:3, :]
    lab01 = jnp.where(p0 >= p1, 0, 1)
    label = jnp.where(jnp.maximum(p0, p1) >= p2, lab01, 2)

    row_cls = jax.lax.broadcasted_iota(jnp.int32, (TOTAL_Q, 1), 0) // QUEUE_SIZE
    mask = row_cls == label

    w = jnp.where(mask, sims + a, 1.0)

    t7 = w[368:375, :]
    a1 = w[0:184, :] * w[184:368, :]
    b1 = a1[0:88, :] * a1[88:176, :]
    r1 = a1[176:184, :]
    c1 = b1[0:40, :] * b1[40:80, :]
    r2 = b1[80:88, :]
    d1 = c1[0:16, :] * c1[16:32, :]
    r3 = c1[32:40, :]
    e1 = d1[0:8, :] * d1[8:16, :]
    f1 = r1 * r2 * r3

    seg = (jnp.sum(jnp.log2(e1), axis=0, keepdims=True)
           + jnp.sum(jnp.log2(f1), axis=0, keepdims=True)
           + jnp.sum(jnp.log2(t7), axis=0, keepdims=True))

    log_d = jnp.log2(d)
    pos_term = jnp.log2(l_pos + a) - log_d
    per = -(seg - QUEUE_SIZE * log_d + pos_term)
    return jnp.sum(per, axis=(0, 1), keepdims=True)


def _loss_kernel(act_hbm, ema_hbm, plabel_hbm, queue_hbm, out_ref,
                 act_v, ema_v, plabel_v, queue_v, sem):
    h = pl.ds(0, HALF)
    h2 = pl.ds(HALF, HALF)
    copies1 = [
        pltpu.make_async_copy(queue_hbm, queue_v, sem.at[0]),
        pltpu.make_async_copy(act_hbm.at[h, :], act_v.at[h, :], sem.at[1]),
        pltpu.make_async_copy(ema_hbm.at[h, :], ema_v.at[h, :], sem.at[2]),
        pltpu.make_async_copy(plabel_hbm, plabel_v, sem.at[3]),
    ]
    copies2 = [
        pltpu.make_async_copy(act_hbm.at[h2, :], act_v.at[h2, :], sem.at[4]),
        pltpu.make_async_copy(ema_hbm.at[h2, :], ema_v.at[h2, :], sem.at[5]),
    ]
    for c in copies1:
        c.start()
    for c in copies2:
        c.start()
    for c in copies1:
        c.wait()

    queue = queue_v[...]
    s1 = _half_loss(act_v[h, :], ema_v[h, :], plabel_v[h, :], queue)

    for c in copies2:
        c.wait()
    s2 = _half_loss(act_v[h2, :], ema_v[h2, :], plabel_v[h2, :], queue)

    out_ref[...] = (s1 + s2) * (LN2 / (BATCH * (QUEUE_SIZE + 1)))


def kernel(activation, ema_activation, pseudo_label, queue_list):
    out = pl.pallas_call(
        _loss_kernel,
        in_specs=[
            pl.BlockSpec(memory_space=pltpu.MemorySpace.HBM),
            pl.BlockSpec(memory_space=pltpu.MemorySpace.HBM),
            pl.BlockSpec(memory_space=pltpu.MemorySpace.HBM),
            pl.BlockSpec(memory_space=pltpu.MemorySpace.HBM),
        ],
        out_shape=jax.ShapeDtypeStruct((1, 1), jnp.float32),
        scratch_shapes=[
            pltpu.VMEM((BATCH, PROJ_DIM), jnp.float32),
            pltpu.VMEM((BATCH, PROJ_DIM), jnp.float32),
            pltpu.VMEM((BATCH, CLASS_NUM), jnp.float32),
            pltpu.VMEM((TOTAL_Q, PROJ_DIM), jnp.float32),
            pltpu.SemaphoreType.DMA((6,)),
        ],
    )(activation, ema_activation, pseudo_label, queue_list)
    return out[0, 0]


# speedup vs baseline: 1.0491x; 1.0491x over previous
import jax
import jax.numpy as jnp
from jax.experimental import pallas as pl

TEMPERATURE = 0.5
QUEUE_SIZE = 125
CLASS_NUM = 3
PROJ_DIM = 128
BATCH = 1024
TOTAL_Q = QUEUE_SIZE * CLASS_NUM
LOG2E = 1.4426950408889634
LN2 = 0.6931471805599453


def _loss_kernel(act_ref, ema_ref, plabel_ref, queue_ref, out_ref):
    act = act_ref[...]
    ema = ema_ref[...]
    pl_probs = plabel_ref[...]
    queue = queue_ref[...]

    def dot_bt(x, y):
        return jax.lax.dot_general(x, y, (((1,), (1,)), ((), ())),
                                   preferred_element_type=jnp.float32)

    ones_k = jnp.ones((1, PROJ_DIM), dtype=jnp.float32)
    eps2 = 1e-24

    n2 = jnp.maximum(dot_bt(ones_k, act * act), eps2)
    ne2 = jnp.maximum(dot_bt(ones_k, ema * ema), eps2)
    s_ae = dot_bt(ones_k, act * ema)

    scale = LOG2E / TEMPERATURE
    rn = jax.lax.rsqrt(n2)
    tau_p = (scale * s_ae) * rn * jax.lax.rsqrt(ne2)
    l_pos = jax.lax.exp2(tau_p)

    tau = dot_bt(queue, act) * (scale * rn)
    sims = jax.lax.exp2(tau)

    total = jnp.sum(sims, axis=0, keepdims=True)
    d = l_pos + total
    a = 1e-6 * d

    plt = dot_bt(jnp.eye(CLASS_NUM, dtype=jnp.float32), pl_probs)
    p0 = plt[0:1, :]
    p1 = plt[1:2, :]
    p2 = plt[2:3, :]
    lab01 = jnp.where(p0 >= p1, 0, 1)
    label = jnp.where(jnp.maximum(p0, p1) >= p2, lab01, 2)

    row_cls = jax.lax.broadcasted_iota(jnp.int32, (TOTAL_Q, 1), 0) // QUEUE_SIZE
    mask = row_cls == label

    w = jnp.where(mask, sims + a, 1.0)

    t7 = w[368:375, :]
    a1 = w[0:184, :] * w[184:368, :]
    b1 = a1[0:88, :] * a1[88:176, :]
    r1 = a1[176:184, :]
    c1 = b1[0:40, :] * b1[40:80, :]
    r2 = b1[80:88, :]
    d1 = c1[0:16, :] * c1[16:32, :]
    r3 = c1[32:40, :]
    e1 = d1[0:8, :] * d1[8:16, :]
    f1 = r1 * r2 * r3

    seg = (jnp.sum(jnp.log2(e1), axis=0, keepdims=True)
           + jnp.sum(jnp.log2(f1), axis=0, keepdims=True)
           + jnp.sum(jnp.log2(t7), axis=0, keepdims=True))

    log_d = jnp.log2(d)
    pos_term = jnp.log2(l_pos + a) - log_d
    per = -(seg - QUEUE_SIZE * log_d + pos_term)
    out_ref[...] = jnp.sum(per, axis=(0, 1), keepdims=True) * (
        LN2 / (BATCH * (QUEUE_SIZE + 1)))


def kernel(activation, ema_activation, pseudo_label, queue_list):
    out = pl.pallas_call(
        _loss_kernel,
        out_shape=jax.ShapeDtypeStruct((1, 1), jnp.float32),
    )(activation, ema_activation, pseudo_label, queue_list)
    return out[0, 0]
